# Initial kernel scaffold; baseline (speedup 1.0000x reference)
import jax
import jax.numpy as jnp
from jax.experimental import pallas as pl


def kernel(scalar_representation, idx_m, W1, b1, W2, b2):
    raise NotImplementedError("write your pallas kernel here")


# speedup vs baseline: 7.1976x; 7.1976x over previous
import functools

import jax
import jax.numpy as jnp
from jax import lax
from jax.experimental import pallas as pl
from jax.experimental.pallas import tpu as pltpu
from jax.experimental.pallas import tpu_sc as plsc

_N = 320000
_D = 128
_M = 10000
_H = 128
_NC = 2
_NS = 16
_NW = _NC * _NS
_RPW = _N // _NW
_CH = 128
_NCH = _RPW // _CH
_TAIL = _RPW - _NCH * _CH
_MROWS = 632
_MP = _MROWS * _NS


def _sc_body(x_hbm, idx_hbm, out_hbm, acc, rows0, rows1, ia0, ia1,
             itail, zbuf, rsem0, rsem1, isem0, isem1):
  c = lax.axis_index("c")
  s = lax.axis_index("s")
  wid = s * _NC + c
  base = wid * _RPW

  rows = (rows0, rows1)
  ia = (ia0, ia1)
  rsem = (rsem0, rsem1)
  isem = (isem0, isem1)

  def fire(j, b):
    off = base + j * _CH
    pltpu.async_copy(x_hbm.at[pl.ds(off, _CH)], rows[b], rsem[b])
    pltpu.async_copy(idx_hbm.at[pl.ds(off, _CH)], ia[b], isem[b])

  def wait(b):
    pltpu.make_async_copy(x_hbm.at[pl.ds(0, _CH)], rows[b], rsem[b]).wait()
    pltpu.make_async_copy(idx_hbm.at[pl.ds(0, _CH)], ia[b], isem[b]).wait()

  def scatter(b):
    pltpu.sync_copy(rows[b], acc.at[ia[b]], add=True)

  fire(0, 0)
  fire(1, 1)

  def zloop(k, carry):
    r = k // 8
    col = (k % 8) * 16
    zbuf[r, pl.ds(col, 16)] = jnp.zeros((16,), jnp.float32)
    return carry
  lax.fori_loop(0, 64 * 8, zloop, 0)
  r0 = s * _MROWS
  for t in range(0, (_MROWS // 64) * 64, 64):
    pltpu.sync_copy(zbuf, acc.at[pl.ds(r0 + t, 64)])
  rem = _MROWS % 64
  if rem:
    pltpu.sync_copy(zbuf.at[pl.ds(0, rem)],
                    acc.at[pl.ds(r0 + (_MROWS // 64) * 64, rem)])
  plsc.subcore_barrier()

  def lbody(jj, carry):
    for b in range(2):
      j = jj * 2 + b
      wait(b)
      scatter(b)
      if b == 0:
        fire(j + 2, b)
      else:
        @pl.when(jj < _NCH // 2 - 1)
        def _():
          fire(j + 2, b)
    return carry
  lax.fori_loop(0, _NCH // 2, lbody, 0)
  wait(0)
  scatter(0)
  if _TAIL:
    toff = base + _NCH * _CH
    pltpu.sync_copy(x_hbm.at[pl.ds(toff, _TAIL)], rows0.at[pl.ds(0, _TAIL)])
    pltpu.sync_copy(idx_hbm.at[pl.ds(toff, _TAIL)], itail)
    pltpu.sync_copy(rows0.at[pl.ds(0, _TAIL)], acc.at[itail], add=True)

  plsc.subcore_barrier()
  pltpu.sync_copy(acc.at[pl.ds(r0, _MROWS)],
                  out_hbm.at[pl.ds(c * _MP + r0, _MROWS)])


_sc_segment_sum = functools.partial(
    pl.kernel,
    out_type=jax.ShapeDtypeStruct((_NC * _MP, _D), jnp.float32),
    mesh=plsc.VectorSubcoreMesh(core_axis_name="c", subcore_axis_name="s"),
    scratch_types=[
        pltpu.VMEM_SHARED((_MP, _D), jnp.float32),
        pltpu.VMEM((_CH, _D), jnp.float32),
        pltpu.VMEM((_CH, _D), jnp.float32),
        pltpu.VMEM((_CH,), jnp.int32),
        pltpu.VMEM((_CH,), jnp.int32),
        pltpu.VMEM((_TAIL,), jnp.int32),
        pltpu.VMEM((64, _D), jnp.float32),
        pltpu.SemaphoreType.DMA,
        pltpu.SemaphoreType.DMA,
        pltpu.SemaphoreType.DMA,
        pltpu.SemaphoreType.DMA,
    ],
)(_sc_body)


_BM = 632


def _tc_body(p_ref, w1_ref, b1_ref, w2_ref, b2_ref, o_ref):
  agg = p_ref[0] + p_ref[1]
  h = jnp.dot(agg, w1_ref[...].T, preferred_element_type=jnp.float32)
  h = h + b1_ref[...]
  h = h * jax.nn.sigmoid(h)
  y = jnp.sum(h * w2_ref[...], axis=1, keepdims=True) + b2_ref[...]
  o_ref[...] = y


_tc_mlp = pl.pallas_call(
    _tc_body,
    grid=(_MP // _BM,),
    in_specs=[
        pl.BlockSpec((2, _BM, _D), lambda i: (0, i, 0)),
        pl.BlockSpec((_H, _D), lambda i: (0, 0)),
        pl.BlockSpec((1, _H), lambda i: (0, 0)),
        pl.BlockSpec((1, _H), lambda i: (0, 0)),
        pl.BlockSpec((1, 1), lambda i: (0, 0)),
    ],
    out_specs=pl.BlockSpec((_BM, 1), lambda i: (i, 0)),
    out_shape=jax.ShapeDtypeStruct((_MP, 1), jnp.float32),
)


def kernel(scalar_representation, idx_m, W1, b1, W2, b2):
  parts = _sc_segment_sum(scalar_representation, idx_m)
  p3 = parts.reshape(_NC, _MP, _D)
  y = _tc_mlp(p3, W1, b1.reshape(1, _H), W2, b2.reshape(1, 1))
  return y[:_M]


# speedup vs baseline: 7.5359x; 1.0470x over previous
import functools

import jax
import jax.numpy as jnp
from jax import lax
from jax.experimental import pallas as pl
from jax.experimental.pallas import tpu as pltpu
from jax.experimental.pallas import tpu_sc as plsc

_N = 320000
_D = 128
_M = 10000
_H = 128
_NC = 2
_NS = 16
_NW = _NC * _NS
_RPW = _N // _NW
_CH = 128
_NCH = _RPW // _CH
_TAIL = _RPW - _NCH * _CH
_MROWS = 632
_MP = _MROWS * _NS


def _sc_body(x_hbm, idx_hbm, out0_hbm, out1_hbm, acc, rows0, rows1, ia0, ia1,
             itail, zbuf, rsem0, rsem1, isem0, isem1):
  c = lax.axis_index("c")
  s = lax.axis_index("s")
  wid = s * _NC + c
  base = wid * _RPW

  rows = (rows0, rows1)
  ia = (ia0, ia1)
  rsem = (rsem0, rsem1)
  isem = (isem0, isem1)

  def fire(j, b):
    off = base + j * _CH
    pltpu.async_copy(x_hbm.at[pl.ds(off, _CH)], rows[b], rsem[b])
    pltpu.async_copy(idx_hbm.at[pl.ds(off, _CH)], ia[b], isem[b])

  def wait(b):
    pltpu.make_async_copy(x_hbm.at[pl.ds(0, _CH)], rows[b], rsem[b]).wait()
    pltpu.make_async_copy(idx_hbm.at[pl.ds(0, _CH)], ia[b], isem[b]).wait()

  def scatter(b):
    pltpu.sync_copy(rows[b], acc.at[ia[b]], add=True)

  fire(0, 0)
  fire(1, 1)

  def zloop(k, carry):
    r = k // 8
    col = (k % 8) * 16
    zbuf[r, pl.ds(col, 16)] = jnp.zeros((16,), jnp.float32)
    return carry
  lax.fori_loop(0, 64 * 8, zloop, 0)
  r0 = s * _MROWS
  for t in range(0, (_MROWS // 64) * 64, 64):
    pltpu.sync_copy(zbuf, acc.at[pl.ds(r0 + t, 64)])
  rem = _MROWS % 64
  if rem:
    pltpu.sync_copy(zbuf.at[pl.ds(0, rem)],
                    acc.at[pl.ds(r0 + (_MROWS // 64) * 64, rem)])
  plsc.subcore_barrier()

  def lbody(jj, carry):
    for b in range(2):
      j = jj * 2 + b
      wait(b)
      scatter(b)
      if b == 0:
        fire(j + 2, b)
      else:
        @pl.when(jj < _NCH // 2 - 1)
        def _():
          fire(j + 2, b)
    return carry
  lax.fori_loop(0, _NCH // 2, lbody, 0)
  wait(0)
  scatter(0)
  if _TAIL:
    toff = base + _NCH * _CH
    pltpu.sync_copy(x_hbm.at[pl.ds(toff, _TAIL)], rows0.at[pl.ds(0, _TAIL)])
    pltpu.sync_copy(idx_hbm.at[pl.ds(toff, _TAIL)], itail)
    pltpu.sync_copy(rows0.at[pl.ds(0, _TAIL)], acc.at[itail], add=True)

  plsc.subcore_barrier()
  @pl.when(c == 0)
  def _():
    pltpu.sync_copy(acc.at[pl.ds(r0, _MROWS)], out0_hbm.at[pl.ds(r0, _MROWS)])
  @pl.when(c == 1)
  def _():
    pltpu.sync_copy(acc.at[pl.ds(r0, _MROWS)], out1_hbm.at[pl.ds(r0, _MROWS)])


_sc_segment_sum = functools.partial(
    pl.kernel,
    out_type=(jax.ShapeDtypeStruct((_MP, _D), jnp.float32),
              jax.ShapeDtypeStruct((_MP, _D), jnp.float32)),
    mesh=plsc.VectorSubcoreMesh(core_axis_name="c", subcore_axis_name="s"),
    scratch_types=[
        pltpu.VMEM_SHARED((_MP, _D), jnp.float32),
        pltpu.VMEM((_CH, _D), jnp.float32),
        pltpu.VMEM((_CH, _D), jnp.float32),
        pltpu.VMEM((_CH,), jnp.int32),
        pltpu.VMEM((_CH,), jnp.int32),
        pltpu.VMEM((_TAIL,), jnp.int32),
        pltpu.VMEM((64, _D), jnp.float32),
        pltpu.SemaphoreType.DMA,
        pltpu.SemaphoreType.DMA,
        pltpu.SemaphoreType.DMA,
        pltpu.SemaphoreType.DMA,
    ],
)(_sc_body)


_BM = 2000


def _tc_body(p0_ref, p1_ref, w1_ref, b1_ref, w2_ref, b2_ref, o_ref):
  agg = p0_ref[...] + p1_ref[...]
  h = jnp.dot(agg, w1_ref[...].T, preferred_element_type=jnp.float32)
  h = h + b1_ref[...]
  h = h * jax.nn.sigmoid(h)
  y = jnp.sum(h * w2_ref[...], axis=1, keepdims=True) + b2_ref[...]
  o_ref[...] = y


_tc_mlp = pl.pallas_call(
    _tc_body,
    grid=(_M // _BM,),
    in_specs=[
        pl.BlockSpec((_BM, _D), lambda i: (i, 0)),
        pl.BlockSpec((_BM, _D), lambda i: (i, 0)),
        pl.BlockSpec((_H, _D), lambda i: (0, 0)),
        pl.BlockSpec((1, _H), lambda i: (0, 0)),
        pl.BlockSpec((1, _H), lambda i: (0, 0)),
        pl.BlockSpec((1, 1), lambda i: (0, 0)),
    ],
    out_specs=pl.BlockSpec((_BM, 1), lambda i: (i, 0)),
    out_shape=jax.ShapeDtypeStruct((_M, 1), jnp.float32),
)


def kernel(scalar_representation, idx_m, W1, b1, W2, b2):
  p0, p1 = _sc_segment_sum(scalar_representation, idx_m)
  return _tc_mlp(p0, p1, W1, b1.reshape(1, _H), W2, b2.reshape(1, 1))


# speedup vs baseline: 7.6138x; 1.0103x over previous
import functools

import jax
import jax.numpy as jnp
from jax import lax
from jax.experimental import pallas as pl
from jax.experimental.pallas import tpu as pltpu
from jax.experimental.pallas import tpu_sc as plsc

_N = 320000
_D = 128
_M = 10000
_H = 128
_NC = 2
_NS = 16
_NW = _NC * _NS
_RPW = _N // _NW
_CH = 128
_NCH = _RPW // _CH
_TAIL = _RPW - _NCH * _CH
_MROWS = 632
_MP = _MROWS * _NS


def _sc_body(x_hbm, idx_hbm, out0_hbm, out1_hbm, acc, rows0, rows1, ia0, ia1,
             itail, idxall, zbuf, rsem0, rsem1, isem0, isem1):
  c = lax.axis_index("c")
  s = lax.axis_index("s")
  wid = s * _NC + c
  base = wid * _RPW

  rows = (rows0, rows1)
  ia = (ia0, ia1)
  rsem = (rsem0, rsem1)
  isem = (isem0, isem1)

  def fire(j, b):
    off = base + j * _CH
    pltpu.async_copy(x_hbm.at[pl.ds(off, _CH)], rows[b], rsem[b])

  def wait(b):
    pltpu.make_async_copy(x_hbm.at[pl.ds(0, _CH)], rows[b], rsem[b]).wait()

  def stage_ids(j, b):
    def cp(k, carry):
      ia[b][pl.ds(k * 16, 16)] = idxall[pl.ds(j * _CH + k * 16, 16)]
      return carry
    lax.fori_loop(0, _CH // 16, cp, 0)

  def scatter(b):
    pltpu.sync_copy(rows[b], acc.at[ia[b]], add=True)

  fire(0, 0)
  fire(1, 1)
  pltpu.async_copy(idx_hbm.at[pl.ds(base, _RPW)], idxall.at[pl.ds(0, _RPW)],
                   isem0)

  def zloop(k, carry):
    r = k // 8
    col = (k % 8) * 16
    zbuf[r, pl.ds(col, 16)] = jnp.zeros((16,), jnp.float32)
    return carry
  lax.fori_loop(0, 32 * 8, zloop, 0)
  r0 = s * _MROWS
  for t in range(0, (_MROWS // 32) * 32, 32):
    pltpu.sync_copy(zbuf, acc.at[pl.ds(r0 + t, 32)])
  rem = _MROWS % 32
  if rem:
    pltpu.sync_copy(zbuf.at[pl.ds(0, rem)],
                    acc.at[pl.ds(r0 + (_MROWS // 32) * 32, rem)])
  plsc.subcore_barrier()

  pltpu.make_async_copy(idx_hbm.at[pl.ds(0, _RPW)], idxall.at[pl.ds(0, _RPW)],
                        isem0).wait()

  def lbody(jj, carry):
    for b in range(2):
      j = jj * 2 + b
      stage_ids(j, b)
      wait(b)
      scatter(b)
      if b == 0:
        fire(j + 2, b)
      else:
        @pl.when(jj < _NCH // 2 - 1)
        def _():
          fire(j + 2, b)
    return carry
  lax.fori_loop(0, _NCH // 2, lbody, 0)
  stage_ids(_NCH - 1, 0)
  wait(0)
  scatter(0)
  if _TAIL:
    toff = base + _NCH * _CH
    pltpu.sync_copy(x_hbm.at[pl.ds(toff, _TAIL)], rows0.at[pl.ds(0, _TAIL)])
    def cpt(k, carry):
      itail[pl.ds(k * 16, 16)] = idxall[pl.ds(_NCH * _CH + k * 16, 16)]
      return carry
    lax.fori_loop(0, _TAIL // 16, cpt, 0)
    pltpu.sync_copy(rows0.at[pl.ds(0, _TAIL)], acc.at[itail], add=True)

  plsc.subcore_barrier()
  @pl.when(c == 0)
  def _():
    pltpu.sync_copy(acc.at[pl.ds(r0, _MROWS)], out0_hbm.at[pl.ds(r0, _MROWS)])
  @pl.when(c == 1)
  def _():
    pltpu.sync_copy(acc.at[pl.ds(r0, _MROWS)], out1_hbm.at[pl.ds(r0, _MROWS)])


_sc_segment_sum = functools.partial(
    pl.kernel,
    out_type=(jax.ShapeDtypeStruct((_MP, _D), jnp.float32),
              jax.ShapeDtypeStruct((_MP, _D), jnp.float32)),
    mesh=plsc.VectorSubcoreMesh(core_axis_name="c", subcore_axis_name="s"),
    scratch_types=[
        pltpu.VMEM_SHARED((_MP, _D), jnp.float32),
        pltpu.VMEM((_CH, _D), jnp.float32),
        pltpu.VMEM((_CH, _D), jnp.float32),
        pltpu.VMEM((_CH,), jnp.int32),
        pltpu.VMEM((_CH,), jnp.int32),
        pltpu.VMEM((_TAIL,), jnp.int32),
        pltpu.VMEM((_RPW + 16,), jnp.int32),
        pltpu.VMEM((32, _D), jnp.float32),
        pltpu.SemaphoreType.DMA,
        pltpu.SemaphoreType.DMA,
        pltpu.SemaphoreType.DMA,
        pltpu.SemaphoreType.DMA,
    ],
)(_sc_body)


_BM = 2000


def _tc_body(p0_ref, p1_ref, w1_ref, b1_ref, w2_ref, b2_ref, o_ref):
  agg = p0_ref[...] + p1_ref[...]
  h = jnp.dot(agg, w1_ref[...].T, preferred_element_type=jnp.float32)
  h = h + b1_ref[...]
  h = h * jax.nn.sigmoid(h)
  y = jnp.sum(h * w2_ref[...], axis=1, keepdims=True) + b2_ref[...]
  o_ref[...] = y


_tc_mlp = pl.pallas_call(
    _tc_body,
    grid=(_M // _BM,),
    in_specs=[
        pl.BlockSpec((_BM, _D), lambda i: (i, 0)),
        pl.BlockSpec((_BM, _D), lambda i: (i, 0)),
        pl.BlockSpec((_H, _D), lambda i: (0, 0)),
        pl.BlockSpec((1, _H), lambda i: (0, 0)),
        pl.BlockSpec((1, _H), lambda i: (0, 0)),
        pl.BlockSpec((1, 1), lambda i: (0, 0)),
    ],
    out_specs=pl.BlockSpec((_BM, 1), lambda i: (i, 0)),
    out_shape=jax.ShapeDtypeStruct((_M, 1), jnp.float32),
)


def kernel(scalar_representation, idx_m, W1, b1, W2, b2):
  p0, p1 = _sc_segment_sum(scalar_representation, idx_m)
  return _tc_mlp(p0, p1, W1, b1.reshape(1, _H), W2, b2.reshape(1, 1))
